# baseline (speedup 1.0000x reference)
import functools

import jax
import jax.numpy as jnp
from jax import lax
from jax.experimental import pallas as pl
from jax.experimental.pallas import tpu as pltpu
from jax.experimental.pallas import tpu_sc as plsc

_NC = 2
_NS = 16
_NW = _NC * _NS
_CROWS = 8
_NBUF = 6


def _sc_body(row0, rows, cols, fp_hbm, x_hbm, out_hbm, fpv, buf, *sems):
    wid = lax.axis_index("s") * _NC + lax.axis_index("c")
    per_w = rows // _NW
    steps = per_w // _CROWS
    base = wid * per_w

    pltpu.sync_copy(fp_hbm, fpv)
    a0 = fpv[0, :]
    a1 = fpv[1, :]
    a2 = fpv[2, :]
    a3 = fpv[3, :]
    b0, b1 = jnp.minimum(a0, a1), jnp.maximum(a0, a1)
    b2, b3 = jnp.minimum(a2, a3), jnp.maximum(a2, a3)
    c0, c2 = jnp.minimum(b0, b2), jnp.maximum(b0, b2)
    c1, c3 = jnp.minimum(b1, b3), jnp.maximum(b1, b3)
    v0, v3 = c0, c3
    v1, v2 = jnp.minimum(c1, c2), jnp.maximum(c1, c2)
    m1 = (v0 + v1) * 0.5
    m2 = (v1 + v2) * 0.5
    m3 = (v2 + v3) * 0.5

    sems_in = sems[:_NBUF]
    sems_out = sems[_NBUF:]

    def in_copy(g, slot):
        return pltpu.make_async_copy(
            x_hbm.at[pl.ds(row0 + base + g * _CROWS, _CROWS)],
            buf.at[slot], sems_in[slot])

    def out_copy(g, slot):
        return pltpu.make_async_copy(
            buf.at[slot], out_hbm.at[pl.ds(base + g * _CROWS, _CROWS)],
            sems_out[slot])

    for s in range(_NBUF):
        in_copy(s, s).start()

    prefetch = _NBUF - 2

    def step(g, slot):
        @pl.when(g >= 2)
        def _():
            out_copy(g - 2, (slot + prefetch) % _NBUF).wait()

            @pl.when(g + prefetch < steps)
            def _():
                in_copy(g + prefetch, (slot + prefetch) % _NBUF).start()

        in_copy(g, slot).wait()
        dat = buf.at[slot]

        for r in range(_CROWS):
            @plsc.parallel_loop(0, cols, step=16, unroll=8)
            def body(i):
                xv = dat[r, pl.ds(i, 16)]
                q = jnp.where(xv > m2,
                              jnp.where(xv > m3, v3, v2),
                              jnp.where(xv > m1, v1, v0))
                dat[r, pl.ds(i, 16)] = q

        out_copy(g, slot).start()

    def rnd(p, _):
        g = p * _NBUF
        for s in range(_NBUF):
            step(g + s, s)
        return 0

    n_full = steps // _NBUF
    lax.fori_loop(0, n_full, rnd, 0)
    for t in range(n_full * _NBUF, steps):
        step(jnp.int32(t), t % _NBUF)
    out_copy(steps - 2, (steps - 2) % _NBUF).wait()
    out_copy(steps - 1, (steps - 1) % _NBUF).wait()


def _sc_quant(fp_bcast, x2, row0, rows):
    cols = x2.shape[1]
    mesh = plsc.VectorSubcoreMesh(core_axis_name="c", subcore_axis_name="s")
    return pl.kernel(
        functools.partial(_sc_body, row0, rows, cols),
        out_type=jax.ShapeDtypeStruct((rows, cols), jnp.float32),
        mesh=mesh,
        scratch_types=(
            [pltpu.VMEM((4, 16), jnp.float32),
             pltpu.VMEM((_NBUF, _CROWS, 2048), jnp.float32)]
            + [pltpu.SemaphoreType.DMA] * (2 * _NBUF)
        ),
    )(fp_bcast, x2)


def kernel(x, fp_values):
    fp_bcast = jnp.asarray(
        jnp.broadcast_to(fp_values.reshape(4, 1), (4, 16)), jnp.float32)
    x2 = x.reshape(-1, x.shape[-1])
    rows = x2.shape[0]
    out = _sc_quant(fp_bcast, x2, 0, rows)
    return out.reshape(x.shape)


# speedup vs baseline: 1.0465x; 1.0465x over previous
import functools

import jax
import jax.numpy as jnp
from jax import lax
from jax.experimental import pallas as pl
from jax.experimental.pallas import tpu as pltpu
from jax.experimental.pallas import tpu_sc as plsc

_NC = 2
_NS = 16
_NW = _NC * _NS
_CROWS = 8
_NBUF = 4


def _sc_body(row0, rows, cols, fp_hbm, x_hbm, out_hbm, fpv, buf, *sems):
    wid = lax.axis_index("c") * _NS + lax.axis_index("s")
    per_w = rows // _NW
    steps = per_w // _CROWS
    base = wid * per_w

    pltpu.sync_copy(fp_hbm, fpv)
    a0 = fpv[0, :]
    a1 = fpv[1, :]
    a2 = fpv[2, :]
    a3 = fpv[3, :]
    b0, b1 = jnp.minimum(a0, a1), jnp.maximum(a0, a1)
    b2, b3 = jnp.minimum(a2, a3), jnp.maximum(a2, a3)
    c0, c2 = jnp.minimum(b0, b2), jnp.maximum(b0, b2)
    c1, c3 = jnp.minimum(b1, b3), jnp.maximum(b1, b3)
    v0, v3 = c0, c3
    v1, v2 = jnp.minimum(c1, c2), jnp.maximum(c1, c2)
    m1 = (v0 + v1) * 0.5
    m2 = (v1 + v2) * 0.5
    m3 = (v2 + v3) * 0.5

    sems_in = sems[:_NBUF]
    sems_out = sems[_NBUF:]

    def in_copy(g, slot):
        return pltpu.make_async_copy(
            x_hbm.at[pl.ds(row0 + base + g * _CROWS, _CROWS)],
            buf.at[slot], sems_in[slot])

    def out_copy(g, slot):
        return pltpu.make_async_copy(
            buf.at[slot], out_hbm.at[pl.ds(base + g * _CROWS, _CROWS)],
            sems_out[slot])

    for s in range(_NBUF):
        in_copy(s, s).start()

    def step(g, slot):
        @pl.when(g >= 2)
        def _():
            out_copy(g - 2, (slot + 2) % _NBUF).wait()

            @pl.when(g + 2 < steps)
            def _():
                in_copy(g + 2, (slot + 2) % _NBUF).start()

        in_copy(g, slot).wait()
        dat = buf.at[slot]

        for r in range(_CROWS):
            @plsc.parallel_loop(0, cols, step=16, unroll=8)
            def body(i):
                xv = dat[r, pl.ds(i, 16)]
                q = jnp.where(xv > m2,
                              jnp.where(xv > m3, v3, v2),
                              jnp.where(xv > m1, v1, v0))
                dat[r, pl.ds(i, 16)] = q

        out_copy(g, slot).start()

    def rnd(p, _):
        g = p * _NBUF
        for s in range(_NBUF):
            step(g + s, s)
        return 0

    lax.fori_loop(0, steps // _NBUF, rnd, 0)
    out_copy(steps - 2, (steps - 2) % _NBUF).wait()
    out_copy(steps - 1, (steps - 1) % _NBUF).wait()


def _sc_quant(fp_bcast, x2, row0, rows):
    cols = x2.shape[1]
    mesh = plsc.VectorSubcoreMesh(core_axis_name="c", subcore_axis_name="s")
    return pl.kernel(
        functools.partial(_sc_body, row0, rows, cols),
        out_type=jax.ShapeDtypeStruct((rows, cols), jnp.float32),
        mesh=mesh,
        scratch_types=(
            [pltpu.VMEM((4, 16), jnp.float32),
             pltpu.VMEM((_NBUF, _CROWS, 2048), jnp.float32)]
            + [pltpu.SemaphoreType.DMA] * (2 * _NBUF)
        ),
    )(fp_bcast, x2)


def kernel(x, fp_values):
    fp_bcast = jnp.asarray(
        jnp.broadcast_to(fp_values.reshape(4, 1), (4, 16)), jnp.float32)
    x2 = x.reshape(-1, x.shape[-1])
    rows = x2.shape[0]
    out = _sc_quant(fp_bcast, x2, 0, rows)
    return out.reshape(x.shape)
